# baseline (speedup 1.0000x reference)
import functools

import jax
import jax.numpy as jnp
from jax import lax
from jax.experimental import pallas as pl
from jax.experimental.pallas import tpu as pltpu
from jax.experimental.pallas import tpu_sc as plsc

B = 16384
F = 26
V = 100000
E = 32
NUM = 13
NUM_P = 16
H1 = 128
H2 = 64
EPS = 1e-5

NC = 2
NS = 16
NW = NC * NS
ROWS = B * F
BPW = B // NW
CH = 128
NG = BPW // CH
EP = 64

_sc_mesh = plsc.VectorSubcoreMesh(core_axis_name="c", subcore_axis_name="s")


@functools.partial(
    pl.kernel,
    out_type=jax.ShapeDtypeStruct((ROWS, EP), jnp.float32),
    mesh=_sc_mesh,
    scratch_types=[
        pltpu.VMEM((F, NG, CH), jnp.int32),
        pltpu.VMEM((F, NG, CH), jnp.int32),
        pltpu.VMEM((NG, CH, EP), jnp.float32),
        [pltpu.SemaphoreType.DMA] * NG,
        [pltpu.SemaphoreType.DMA] * NG,
    ],
    compiler_params=pltpu.CompilerParams(use_tc_tiling_on_sc=False),
)
def _sc_gather(tbl_hbm, vidx_hbm, oidx_hbm, out_hbm, vidx_v, oidx_v, rows_v,
               gsems, ssems):
    wid = lax.axis_index("s") * NC + lax.axis_index("c")
    pltpu.sync_copy(vidx_hbm.at[wid], vidx_v)
    pltpu.sync_copy(oidx_hbm.at[wid], oidx_v)

    def fchunk(f, carry):
        @pl.when(f > 0)
        def _():
            for g in range(NG):
                pltpu.make_async_copy(
                    rows_v.at[g], out_hbm.at[oidx_v.at[f - 1, g]],
                    ssems[g]).wait()

        hg = [
            pltpu.async_copy(
                tbl_hbm.at[f].at[vidx_v.at[f, g]], rows_v.at[g], gsems[g])
            for g in range(NG)
        ]
        for g in range(NG):
            hg[g].wait()
            pltpu.async_copy(
                rows_v.at[g], out_hbm.at[oidx_v.at[f, g]], ssems[g])
        return carry

    lax.fori_loop(0, F, fchunk, 0)
    for g in range(NG):
        pltpu.make_async_copy(
            rows_v.at[g], out_hbm.at[oidx_v.at[F - 1, g]], ssems[g]).wait()


BLK = 1024
NB = B // BLK


def _mlp_body(emb_ref, num_ref, w1e_ref, w1n_ref, b1_ref, g1_ref, be1_ref,
              w2_ref, b2_ref, g2_ref, be2_ref, wo_ref, bo_ref,
              out_ref, h1_acc, h2_acc, st1, st2):
    p = pl.program_id(0)
    j = pl.program_id(1)
    hp = lax.Precision.HIGHEST

    @pl.when(p == 0)
    def _layer1():
        h = jnp.dot(emb_ref[...], w1e_ref[...],
                    preferred_element_type=jnp.float32, precision=hp)
        h = h + jnp.dot(num_ref[...], w1n_ref[...],
                        preferred_element_type=jnp.float32, precision=hp)
        h = h + b1_ref[...]
        h1_acc[pl.ds(j * BLK, BLK), :] = h
        s = jnp.sum(h, axis=0, keepdims=True)
        ss = jnp.sum(h * h, axis=0, keepdims=True)

        @pl.when(j == 0)
        def _():
            st1[0:1, :] = s
            st1[1:2, :] = ss

        @pl.when(j > 0)
        def _():
            st1[0:1, :] += s
            st1[1:2, :] += ss

    @pl.when(p == 1)
    def _layer2():
        m1 = st1[0:1, :] * (1.0 / B)
        v1 = st1[1:2, :] * (1.0 / B) - m1 * m1
        hb = h1_acc[pl.ds(j * BLK, BLK), :]
        h1n = jnp.maximum(
            (hb - m1) * lax.rsqrt(v1 + EPS) * g1_ref[...] + be1_ref[...], 0.0)
        h2 = jnp.dot(h1n, w2_ref[...],
                     preferred_element_type=jnp.float32, precision=hp)
        h2 = h2 + b2_ref[...]
        h2_acc[pl.ds(j * BLK, BLK), :] = h2
        s = jnp.sum(h2, axis=0, keepdims=True)
        ss = jnp.sum(h2 * h2, axis=0, keepdims=True)

        @pl.when(j == 0)
        def _():
            st2[0:1, :] = s
            st2[1:2, :] = ss

        @pl.when(j > 0)
        def _():
            st2[0:1, :] += s
            st2[1:2, :] += ss

    @pl.when(p == 2)
    def _layer3():
        m2 = st2[0:1, :] * (1.0 / B)
        v2 = st2[1:2, :] * (1.0 / B) - m2 * m2
        hb = h2_acc[pl.ds(j * BLK, BLK), :]
        h2n = jnp.maximum(
            (hb - m2) * lax.rsqrt(v2 + EPS) * g2_ref[...] + be2_ref[...], 0.0)
        out_ref[...] = (jnp.sum(h2n * wo_ref[...], axis=1, keepdims=True)
                        + bo_ref[...])


def _emb_map(p, j):
    return (jnp.where(p == 0, j, 0), 0)


_mlp = pl.pallas_call(
    _mlp_body,
    grid=(3, NB),
    in_specs=[
        pl.BlockSpec((BLK, F * EP), _emb_map),
        pl.BlockSpec((BLK, NUM_P), _emb_map),
        pl.BlockSpec((F * EP, H1), lambda p, j: (0, 0)),
        pl.BlockSpec((NUM_P, H1), lambda p, j: (0, 0)),
        pl.BlockSpec((1, H1), lambda p, j: (0, 0)),
        pl.BlockSpec((1, H1), lambda p, j: (0, 0)),
        pl.BlockSpec((1, H1), lambda p, j: (0, 0)),
        pl.BlockSpec((H1, H2), lambda p, j: (0, 0)),
        pl.BlockSpec((1, H2), lambda p, j: (0, 0)),
        pl.BlockSpec((1, H2), lambda p, j: (0, 0)),
        pl.BlockSpec((1, H2), lambda p, j: (0, 0)),
        pl.BlockSpec((1, H2), lambda p, j: (0, 0)),
        pl.BlockSpec((1, 1), lambda p, j: (0, 0)),
    ],
    out_specs=pl.BlockSpec((BLK, 1), lambda p, j: (j, 0)),
    out_shape=jax.ShapeDtypeStruct((B, 1), jnp.float32),
    scratch_shapes=[
        pltpu.VMEM((B, H1), jnp.float32),
        pltpu.VMEM((B, H2), jnp.float32),
        pltpu.VMEM((2, H1), jnp.float32),
        pltpu.VMEM((2, H2), jnp.float32),
    ],
)


def kernel(categorical_data, numerical_data, tables, W1, b1, g1, be1,
           W2, b2, g2, be2, Wout, bout):
    cat = categorical_data.astype(jnp.int32)
    vidx = cat.reshape(NW, NG, CH, F).transpose(0, 3, 1, 2)
    orow = (jnp.arange(B, dtype=jnp.int32) * F)[:, None] \
        + jnp.arange(F, dtype=jnp.int32)[None, :]
    oidx = orow.reshape(NW, NG, CH, F).transpose(0, 3, 1, 2)
    tbl64 = jnp.pad(tables, ((0, 0), (0, 0), (0, EP - E)))
    emb = _sc_gather(tbl64, vidx, oidx)
    emb2 = emb.reshape(B, F * EP)

    num_p = jnp.pad(numerical_data, ((0, 0), (0, NUM_P - NUM)))
    w1e = jnp.pad(W1[:, :F * E].T.reshape(F, E, H1),
                  ((0, 0), (0, EP - E), (0, 0))).reshape(F * EP, H1)
    w1n = jnp.pad(W1[:, F * E:], ((0, 0), (0, NUM_P - NUM))).T
    out = _mlp(emb2, num_p,
               w1e, w1n, b1[None, :], g1[None, :], be1[None, :],
               W2.T, b2[None, :], g2[None, :], be2[None, :],
               Wout, bout[None, :])
    return out


# speedup vs baseline: 1.7845x; 1.7845x over previous
import functools

import jax
import jax.numpy as jnp
from jax import lax
from jax.experimental import pallas as pl
from jax.experimental.pallas import tpu as pltpu
from jax.experimental.pallas import tpu_sc as plsc

B = 16384
F = 26
V = 100000
E = 32
NUM = 13
NUM_P = 16
H1 = 128
H2 = 64
EPS = 1e-5

NC = 2
NS = 16
NW = NC * NS
ROWS = B * F
BPW = B // NW
CH = 128
NG = BPW // CH

_sc_mesh = plsc.VectorSubcoreMesh(core_axis_name="c", subcore_axis_name="s")


@functools.partial(
    pl.kernel,
    out_type=jax.ShapeDtypeStruct((ROWS, E), jnp.float32),
    mesh=_sc_mesh,
    scratch_types=[
        pltpu.VMEM((F, NG, CH), jnp.int32),
        pltpu.VMEM((F, NG, CH), jnp.int32),
        pltpu.VMEM((NG, CH, E), jnp.float32),
        [pltpu.SemaphoreType.DMA] * NG,
        [pltpu.SemaphoreType.DMA] * NG,
    ],
    compiler_params=pltpu.CompilerParams(use_tc_tiling_on_sc=False),
)
def _sc_gather(tbl_hbm, vidx_hbm, oidx_hbm, out_hbm, vidx_v, oidx_v, rows_v,
               gsems, ssems):
    wid = lax.axis_index("s") * NC + lax.axis_index("c")
    pltpu.sync_copy(vidx_hbm.at[wid], vidx_v)
    pltpu.sync_copy(oidx_hbm.at[wid], oidx_v)

    def fchunk(f, carry):
        @pl.when(f > 0)
        def _():
            for g in range(NG):
                pltpu.make_async_copy(
                    rows_v.at[g], out_hbm.at[oidx_v.at[f - 1, g]],
                    ssems[g]).wait()

        hg = [
            pltpu.async_copy(
                tbl_hbm.at[f].at[vidx_v.at[f, g]], rows_v.at[g], gsems[g])
            for g in range(NG)
        ]
        for g in range(NG):
            hg[g].wait()
            pltpu.async_copy(
                rows_v.at[g], out_hbm.at[oidx_v.at[f, g]], ssems[g])
        return carry

    lax.fori_loop(0, F, fchunk, 0)
    for g in range(NG):
        pltpu.make_async_copy(
            rows_v.at[g], out_hbm.at[oidx_v.at[F - 1, g]], ssems[g]).wait()


BLK = 2048
NB = B // BLK


def _mlp_body(emb_ref, num_ref, w1e_ref, w1n_ref, b1_ref, g1_ref, be1_ref,
              w2_ref, b2_ref, g2_ref, be2_ref, wo_ref, bo_ref,
              out_ref, h1_acc, h2_acc, st1, st2):
    p = pl.program_id(0)
    j = pl.program_id(1)
    hp = lax.Precision.HIGHEST

    @pl.when(p == 0)
    def _layer1():
        h = jnp.dot(emb_ref[...], w1e_ref[...],
                    preferred_element_type=jnp.float32, precision=hp)
        h = h + jnp.dot(num_ref[...], w1n_ref[...],
                        preferred_element_type=jnp.float32, precision=hp)
        h = h + b1_ref[...]
        h1_acc[pl.ds(j * BLK, BLK), :] = h
        s = jnp.sum(h, axis=0, keepdims=True)
        ss = jnp.sum(h * h, axis=0, keepdims=True)

        @pl.when(j == 0)
        def _():
            st1[0:1, :] = s
            st1[1:2, :] = ss

        @pl.when(j > 0)
        def _():
            st1[0:1, :] += s
            st1[1:2, :] += ss

    @pl.when(p == 1)
    def _layer2():
        m1 = st1[0:1, :] * (1.0 / B)
        v1 = st1[1:2, :] * (1.0 / B) - m1 * m1
        hb = h1_acc[pl.ds(j * BLK, BLK), :]
        h1n = jnp.maximum(
            (hb - m1) * lax.rsqrt(v1 + EPS) * g1_ref[...] + be1_ref[...], 0.0)
        h2 = jnp.dot(h1n, w2_ref[...],
                     preferred_element_type=jnp.float32, precision=hp)
        h2 = h2 + b2_ref[...]
        h2_acc[pl.ds(j * BLK, BLK), :] = h2
        s = jnp.sum(h2, axis=0, keepdims=True)
        ss = jnp.sum(h2 * h2, axis=0, keepdims=True)

        @pl.when(j == 0)
        def _():
            st2[0:1, :] = s
            st2[1:2, :] = ss

        @pl.when(j > 0)
        def _():
            st2[0:1, :] += s
            st2[1:2, :] += ss

    @pl.when(p == 2)
    def _layer3():
        m2 = st2[0:1, :] * (1.0 / B)
        v2 = st2[1:2, :] * (1.0 / B) - m2 * m2
        hb = h2_acc[pl.ds(j * BLK, BLK), :]
        h2n = jnp.maximum(
            (hb - m2) * lax.rsqrt(v2 + EPS) * g2_ref[...] + be2_ref[...], 0.0)
        out_ref[...] = (jnp.sum(h2n * wo_ref[...], axis=1, keepdims=True)
                        + bo_ref[...])


def _emb_map(p, j):
    return (jnp.where(p == 0, j, 0), 0)


_mlp = pl.pallas_call(
    _mlp_body,
    grid=(3, NB),
    in_specs=[
        pl.BlockSpec((BLK, F * E), _emb_map),
        pl.BlockSpec((BLK, NUM_P), _emb_map),
        pl.BlockSpec((F * E, H1), lambda p, j: (0, 0)),
        pl.BlockSpec((NUM_P, H1), lambda p, j: (0, 0)),
        pl.BlockSpec((1, H1), lambda p, j: (0, 0)),
        pl.BlockSpec((1, H1), lambda p, j: (0, 0)),
        pl.BlockSpec((1, H1), lambda p, j: (0, 0)),
        pl.BlockSpec((H1, H2), lambda p, j: (0, 0)),
        pl.BlockSpec((1, H2), lambda p, j: (0, 0)),
        pl.BlockSpec((1, H2), lambda p, j: (0, 0)),
        pl.BlockSpec((1, H2), lambda p, j: (0, 0)),
        pl.BlockSpec((1, H2), lambda p, j: (0, 0)),
        pl.BlockSpec((1, 1), lambda p, j: (0, 0)),
    ],
    out_specs=pl.BlockSpec((BLK, 1), lambda p, j: (j, 0)),
    out_shape=jax.ShapeDtypeStruct((B, 1), jnp.float32),
    scratch_shapes=[
        pltpu.VMEM((B, H1), jnp.float32),
        pltpu.VMEM((B, H2), jnp.float32),
        pltpu.VMEM((2, H1), jnp.float32),
        pltpu.VMEM((2, H2), jnp.float32),
    ],
)


def kernel(categorical_data, numerical_data, tables, W1, b1, g1, be1,
           W2, b2, g2, be2, Wout, bout):
    cat = categorical_data.astype(jnp.int32)
    vidx = cat.reshape(NW, NG, CH, F).transpose(0, 3, 1, 2)
    orow = (jnp.arange(B, dtype=jnp.int32) * F)[:, None] \
        + jnp.arange(F, dtype=jnp.int32)[None, :]
    oidx = orow.reshape(NW, NG, CH, F).transpose(0, 3, 1, 2)
    emb = _sc_gather(tables, vidx, oidx)
    emb2 = emb.reshape(B, F * E)

    num_p = jnp.pad(numerical_data, ((0, 0), (0, NUM_P - NUM)))
    w1e = W1[:, :F * E].T
    w1n = jnp.pad(W1[:, F * E:], ((0, 0), (0, NUM_P - NUM))).T
    out = _mlp(emb2, num_p,
               w1e, w1n, b1[None, :], g1[None, :], be1[None, :],
               W2.T, b2[None, :], g2[None, :], be2[None, :],
               Wout, bout[None, :])
    return out
